# baseline (speedup 1.0000x reference)
import functools

import jax
import jax.numpy as jnp
from jax.experimental import pallas as pl
from jax.experimental.pallas import tpu as pltpu


def _body(x_ref, adj_top_ref, adj_bot_ref, w_ref, out_ref):
    x = x_ref[...]
    w = w_ref[...]
    h0 = jnp.dot(adj_top_ref[...], x, preferred_element_type=jnp.float32)
    out_ref[0] = jax.lax.dot_general(
        h0, w, (((1,), (1,)), ((), ())), preferred_element_type=jnp.float32)
    h1 = jnp.dot(adj_bot_ref[...], x, preferred_element_type=jnp.float32)
    out_ref[1] = jax.lax.dot_general(
        h1, w, (((1,), (1,)), ((), ())), preferred_element_type=jnp.float32)


@functools.partial(jax.jit, static_argnames=("m_blk", "interpret"))
def _graph_conv(x, adj, W, *, m_blk, interpret=False):
    n, d_in = x.shape
    d_out = W.shape[0]
    half_blocks = (n // 2) // m_blk
    out2 = pl.pallas_call(
        _body,
        grid=(half_blocks,),
        in_specs=[
            pl.BlockSpec((n, d_in), lambda m: (0, 0)),
            pl.BlockSpec((m_blk, n), lambda m: (m, 0)),
            pl.BlockSpec((m_blk, n),
                         lambda m, _hb=half_blocks: (m + _hb, 0)),
            pl.BlockSpec((d_out, d_in), lambda m: (0, 0)),
        ],
        out_specs=pl.BlockSpec((2, m_blk, d_out), lambda m: (0, m, 0)),
        out_shape=jax.ShapeDtypeStruct((2, n // 2, d_out), jnp.float32),
        compiler_params=pltpu.CompilerParams(
            vmem_limit_bytes=64 * 1024 * 1024),
        interpret=interpret,
    )(x, adj, adj, W)
    return out2.reshape(n, d_out)


def kernel(x, adj, W):
    n = x.shape[0]
    if (n // 2) % 200 == 0:
        return _graph_conv(x, adj, W, m_blk=200)
    return _graph_conv(x, adj, W, m_blk=n // 2)


# speedup vs baseline: 1.1125x; 1.1125x over previous
import functools

import jax
import jax.numpy as jnp
from jax.experimental import pallas as pl
from jax.experimental.pallas import tpu as pltpu


def _body(x_ref, adj_ref, w_ref, out_ref):
    h = jnp.dot(adj_ref[...], x_ref[...], preferred_element_type=jnp.float32)
    out_ref[...] = jax.lax.dot_general(
        h, w_ref[...], (((1,), (1,)), ((), ())),
        preferred_element_type=jnp.float32,
    )


@functools.partial(jax.jit, static_argnames=("m_blk", "lookahead", "interpret"))
def _graph_conv(x, adj, W, *, m_blk, lookahead=False, interpret=False):
    n, d_in = x.shape
    d_out = W.shape[0]
    adj_mode = (pl.Buffered(buffer_count=2, use_lookahead=True)
                if lookahead else None)
    return pl.pallas_call(
        _body,
        grid=(n // m_blk,),
        in_specs=[
            pl.BlockSpec((n, d_in), lambda m: (0, 0)),
            pl.BlockSpec((m_blk, n), lambda m: (m, 0),
                         pipeline_mode=adj_mode),
            pl.BlockSpec((d_out, d_in), lambda m: (0, 0)),
        ],
        out_specs=pl.BlockSpec((m_blk, d_out), lambda m: (m, 0)),
        out_shape=jax.ShapeDtypeStruct((n, d_out), jnp.float32),
        compiler_params=pltpu.CompilerParams(
            vmem_limit_bytes=64 * 1024 * 1024),
        interpret=interpret,
    )(x, adj, W)


def kernel(x, adj, W):
    n = x.shape[0]
    m_blk = 400 if n % 400 == 0 else n
    return _graph_conv(x, adj, W, m_blk=m_blk, lookahead=False)
